# Initial kernel scaffold; baseline (speedup 1.0000x reference)
import jax
import jax.numpy as jnp
from jax.experimental import pallas as pl


def kernel(x, edge_index, edge_weight, current_node, W1, b1, W2, b2, W3, b3, W4, b4, W5, b5, Wp, bp, Wv, bv):
    raise NotImplementedError("write your pallas kernel here")


# speedup vs baseline: 9.2025x; 9.2025x over previous
import functools

import jax
import jax.numpy as jnp
from jax import lax
from jax.experimental import pallas as pl
from jax.experimental.pallas import tpu as pltpu
from jax.experimental.pallas import tpu_sc as plsc

N = 10000
E = 160000
D_IN = 256
H = 32
O = 10000

NC = 2
NS = 16
NW = NC * NS
CHUNK = 128
KCH = 40
EPW = KCH * CHUNK
EPAD = EPW * NW
ECH = EPAD // CHUNK
ND = 10240
RPT = ND // NS
NBUF = 4

_mesh = plsc.VectorSubcoreMesh(
    core_axis_name="c", subcore_axis_name="s", num_cores=NC, num_subcores=NS
)


def _prep_body(src_h, dst_h, ew_h, cur_h, z1_h, degp, cntp, mskp,
               sbuf, dbuf, ebuf, cbuf, obuf, hbuf, accd, accc, accm):
    c = lax.axis_index("c")
    s = lax.axis_index("s")
    wid = c * NS + s
    base = wid * KCH
    pltpu.sync_copy(src_h.at[pl.ds(base, KCH)], sbuf)
    pltpu.sync_copy(dst_h.at[pl.ds(base, KCH)], dbuf)
    pltpu.sync_copy(ew_h.at[pl.ds(base, KCH)], ebuf)
    pltpu.sync_copy(cur_h, cbuf)
    off = s * RPT
    pltpu.sync_copy(z1_h.at[pl.ds(off, RPT)], accd.at[pl.ds(off, RPT)])
    pltpu.sync_copy(z1_h.at[pl.ds(off, RPT)], accc.at[pl.ds(off, RPT)])
    pltpu.sync_copy(z1_h.at[pl.ds(off, RPT)], accm.at[pl.ds(off, RPT)])

    def fill_ones(k, carry):
        obuf[pl.ds(k * 16, 16)] = jnp.full((16,), 1.0, jnp.float32)
        return carry

    lax.fori_loop(0, CHUNK // 16, fill_ones, 0)
    plsc.subcore_barrier()
    cv = cbuf[...]

    def chunk(j, carry):
        def hk(k, inner):
            s16 = sbuf[j, pl.ds(k * 16, 16)]
            hbuf[pl.ds(k * 16, 16)] = jnp.where(
                s16 == cv, jnp.full((16,), 1.0, jnp.float32),
                jnp.full((16,), 0.0, jnp.float32))
            return inner

        lax.fori_loop(0, CHUNK // 16, hk, 0)
        pltpu.sync_copy(ebuf.at[j], accd.at[dbuf.at[j]], add=True)
        pltpu.sync_copy(obuf, accc.at[dbuf.at[j]], add=True)
        pltpu.sync_copy(hbuf, accm.at[dbuf.at[j]], add=True)
        return carry

    lax.fori_loop(0, KCH, chunk, 0)
    plsc.subcore_barrier()
    pltpu.sync_copy(accd.at[pl.ds(off, RPT)], degp.at[c, pl.ds(off, RPT)])
    pltpu.sync_copy(accc.at[pl.ds(off, RPT)], cntp.at[c, pl.ds(off, RPT)])
    pltpu.sync_copy(accm.at[pl.ds(off, RPT)], mskp.at[c, pl.ds(off, RPT)])


_prep = pl.kernel(
    _prep_body,
    out_type=(
        jax.ShapeDtypeStruct((NC, ND), jnp.float32),
        jax.ShapeDtypeStruct((NC, ND), jnp.float32),
        jax.ShapeDtypeStruct((NC, ND), jnp.float32),
    ),
    mesh=_mesh,
    scratch_types=[
        pltpu.VMEM((KCH, CHUNK), jnp.int32),
        pltpu.VMEM((KCH, CHUNK), jnp.int32),
        pltpu.VMEM((KCH, CHUNK), jnp.float32),
        pltpu.VMEM((16,), jnp.int32),
        pltpu.VMEM((CHUNK,), jnp.float32),
        pltpu.VMEM((CHUNK,), jnp.float32),
        pltpu.VMEM_SHARED((ND,), jnp.float32),
        pltpu.VMEM_SHARED((ND,), jnp.float32),
        pltpu.VMEM_SHARED((ND,), jnp.float32),
    ],
)


def _agg_body(weighted, src_h, dst_h, ew_h, y_h, z2_h, outp,
              sbuf, dbuf, ebuf, rows, acc, g0, g1, g2, g3):
    gs = (g0, g1, g2, g3)
    c = lax.axis_index("c")
    s = lax.axis_index("s")
    wid = c * NS + s
    base = wid * KCH
    pltpu.sync_copy(src_h.at[pl.ds(base, KCH)], sbuf)
    pltpu.sync_copy(dst_h.at[pl.ds(base, KCH)], dbuf)
    if weighted:
        pltpu.sync_copy(ew_h.at[pl.ds(base, KCH)], ebuf)
    off = s * RPT
    pltpu.sync_copy(z2_h.at[pl.ds(off, RPT)], acc.at[pl.ds(off, RPT)])
    plsc.subcore_barrier()
    for b in range(NBUF):
        pltpu.async_copy(y_h.at[sbuf.at[b]], rows.at[b], gs[b])

    def outer(t, carry):
        for b in range(NBUF):
            j = t * NBUF + b
            pltpu.make_async_copy(y_h.at[sbuf.at[b]], rows.at[b], gs[b]).wait()
            if weighted:
                def esc(k, inner):
                    wv = ebuf[j, pl.ds(k * 16, 16)]
                    for m in range(16):
                        i = k * 16 + m
                        w = wv[m]
                        rows[b, i, pl.ds(0, 16)] = rows[b, i, pl.ds(0, 16)] * w
                        rows[b, i, pl.ds(16, 16)] = (
                            rows[b, i, pl.ds(16, 16)] * w)
                    return inner

                lax.fori_loop(0, CHUNK // 16, esc, 0)
            pltpu.sync_copy(rows.at[b], acc.at[dbuf.at[j]], add=True)

            @pl.when(j + NBUF < KCH)
            def _():
                pltpu.async_copy(y_h.at[sbuf.at[j + NBUF]], rows.at[b], gs[b])

        return carry

    lax.fori_loop(0, KCH // NBUF, outer, 0)
    plsc.subcore_barrier()
    pltpu.sync_copy(acc.at[pl.ds(off, RPT)], outp.at[c, pl.ds(off, RPT)])


def _make_agg(weighted):
    return pl.kernel(
        functools.partial(_agg_body, weighted),
        out_type=jax.ShapeDtypeStruct((NC, ND, H), jnp.float32),
        mesh=_mesh,
        scratch_types=[
            pltpu.VMEM((KCH, CHUNK), jnp.int32),
            pltpu.VMEM((KCH, CHUNK), jnp.int32),
            pltpu.VMEM((KCH, CHUNK), jnp.float32),
            pltpu.VMEM((NBUF, CHUNK, H), jnp.float32),
            pltpu.VMEM_SHARED((ND, H), jnp.float32),
            pltpu.SemaphoreType.DMA,
            pltpu.SemaphoreType.DMA,
            pltpu.SemaphoreType.DMA,
            pltpu.SemaphoreType.DMA,
        ],
        compiler_params=pltpu.CompilerParams(use_tc_tiling_on_sc=False),
    )


_agg_w = _make_agg(True)
_agg_u = _make_agg(False)

RB = 256
GRID = ND // RB
RH = 8


def _k1_body(x_ref, w_ref, p0_ref, p1_ref, o_ref):
    dinv = lax.rsqrt(p0_ref[...] + p1_ref[...] + 1.0)
    o_ref[...] = jnp.dot(
        x_ref[...], w_ref[...], preferred_element_type=jnp.float32) * dinv


def _mid_body(z0_ref, z1_ref, y_ref, pp0_ref, pp1_ref, nn0_ref, nn1_ref,
              b_ref, w_ref, o_ref):
    dinvp = lax.rsqrt(pp0_ref[...] + pp1_ref[...] + 1.0)
    h = jnp.maximum(
        (z0_ref[...] + z1_ref[...] + y_ref[...]) * dinvp + b_ref[...], 0.0)
    dinvn = lax.rsqrt(nn0_ref[...] + nn1_ref[...] + 1.0)
    o_ref[...] = jnp.dot(
        h, w_ref[...], preferred_element_type=jnp.float32) * dinvn


def _head_body(z0_ref, z1_ref, y_ref, cp0_ref, cp1_ref, b5_ref,
               m0_ref, m1_ref, wp_ref, bp_ref, wv_ref, bv_ref,
               p_ref, v_ref):
    dinv2 = lax.rsqrt(cp0_ref[...] + cp1_ref[...] + 1.0)
    h = jnp.maximum(
        (z0_ref[...] + z1_ref[...] + y_ref[...]) * dinv2 + b5_ref[...], 0.0)
    msum = m0_ref[...] + m1_ref[...]
    maskf = jnp.where(msum > 0.0, 1.0, 0.0)
    anyrow = jnp.any(msum > 0.0)

    @pl.when(anyrow)
    def _():
        zp = jnp.dot(h, wp_ref[...], preferred_element_type=jnp.float32)
        zp = zp + bp_ref[...]
        zmax = jnp.max(zp, axis=1, keepdims=True)
        ez = jnp.exp(zp - zmax)
        ssum = jnp.sum(ez, axis=1, keepdims=True)
        p_ref[...] = ez * (maskf / ssum)
        zv = jnp.dot(h, wv_ref[...], preferred_element_type=jnp.float32)
        v_ref[...] = (zv + bv_ref[...]) * maskf

    @pl.when(jnp.logical_not(anyrow))
    def _():
        p_ref[...] = jnp.zeros((RH, O), jnp.float32)
        v_ref[...] = jnp.zeros((RH, O), jnp.float32)


def _row_spec(rows, cols):
    return pl.BlockSpec((rows, cols), lambda i: (i, 0))


def _const_spec(rows, cols):
    return pl.BlockSpec((rows, cols), lambda i: (0, 0))


def kernel(x, edge_index, edge_weight, current_node,
           W1, b1, W2, b2, W3, b3, W4, b4, W5, b5, Wp, bp, Wv, bv):
    src = edge_index[0]
    dst = edge_index[1]
    pad = EPAD - E
    srcp = jnp.concatenate(
        [src, jnp.full((pad,), N, jnp.int32)]).reshape(ECH, CHUNK)
    dstp = jnp.concatenate(
        [dst, jnp.full((pad,), N, jnp.int32)]).reshape(ECH, CHUNK)
    ewp = jnp.concatenate(
        [edge_weight, jnp.zeros((pad,), jnp.float32)]).reshape(ECH, CHUNK)
    cur16 = jnp.full((16,), current_node, jnp.int32)
    z1d = jnp.zeros((ND,), jnp.float32)
    z2d = jnp.zeros((ND, H), jnp.float32)
    x_p = jnp.pad(x, ((0, ND - N), (0, 0)))

    degp, cntp, mskp = _prep(srcp, dstp, ewp, cur16, z1d)
    d10 = degp[0].reshape(ND, 1)
    d11 = degp[1].reshape(ND, 1)
    c0 = cntp[0].reshape(ND, 1)
    c1 = cntp[1].reshape(ND, 1)
    m0 = mskp[0].reshape(ND, 1)
    m1 = mskp[1].reshape(ND, 1)

    y1 = pl.pallas_call(
        _k1_body,
        grid=(GRID,),
        in_specs=[
            _row_spec(RB, D_IN),
            _const_spec(D_IN, H),
            _row_spec(RB, 1),
            _row_spec(RB, 1),
        ],
        out_specs=_row_spec(RB, H),
        out_shape=jax.ShapeDtypeStruct((ND, H), jnp.float32),
    )(x_p, W1, d10, d11)

    def mid(zp, y_prev, pp0, pp1, b_prev, w_next):
        z0 = zp[0]
        z1 = zp[1]
        return pl.pallas_call(
            _mid_body,
            grid=(GRID,),
            in_specs=[
                _row_spec(RB, H),
                _row_spec(RB, H),
                _row_spec(RB, H),
                _row_spec(RB, 1),
                _row_spec(RB, 1),
                _row_spec(RB, 1),
                _row_spec(RB, 1),
                _const_spec(1, H),
                _const_spec(H, H),
            ],
            out_specs=_row_spec(RB, H),
            out_shape=jax.ShapeDtypeStruct((ND, H), jnp.float32),
        )(z0, z1, y_prev, pp0, pp1, c0, c1, b_prev.reshape(1, H), w_next)

    zp1 = _agg_w(srcp, dstp, ewp, y1, z2d)
    y2 = mid(zp1, y1, d10, d11, b1, W2)
    zp2 = _agg_u(srcp, dstp, ewp, y2, z2d)
    y3 = mid(zp2, y2, c0, c1, b2, W3)
    zp3 = _agg_u(srcp, dstp, ewp, y3, z2d)
    y4 = mid(zp3, y3, c0, c1, b3, W4)
    zp4 = _agg_u(srcp, dstp, ewp, y4, z2d)
    y5 = mid(zp4, y4, c0, c1, b4, W5)
    zp5 = _agg_u(srcp, dstp, ewp, y5, z2d)

    p, v = pl.pallas_call(
        _head_body,
        grid=(N // RH,),
        in_specs=[
            _row_spec(RH, H),
            _row_spec(RH, H),
            _row_spec(RH, H),
            _row_spec(RH, 1),
            _row_spec(RH, 1),
            _const_spec(1, H),
            _row_spec(RH, 1),
            _row_spec(RH, 1),
            _const_spec(H, O),
            _const_spec(1, O),
            _const_spec(H, O),
            _const_spec(1, O),
        ],
        out_specs=[_row_spec(RH, O), _row_spec(RH, O)],
        out_shape=[
            jax.ShapeDtypeStruct((N, O), jnp.float32),
            jax.ShapeDtypeStruct((N, O), jnp.float32),
        ],
    )(zp5[0], zp5[1], y5, c0, c1, b5.reshape(1, H), m0, m1,
      Wp, bp.reshape(1, O), Wv, bv.reshape(1, O))
    return (p, v)


# speedup vs baseline: 14.1038x; 1.5326x over previous
import functools

import jax
import jax.numpy as jnp
from jax import lax
from jax.experimental import pallas as pl
from jax.experimental.pallas import tpu as pltpu
from jax.experimental.pallas import tpu_sc as plsc

N = 10000
E = 160000
D_IN = 256
H = 32
O = 10000

NC = 2
NS = 16
NW = NC * NS
CHUNK = 128
KCH = 40
EPW = KCH * CHUNK
EPAD = EPW * NW
ECH = EPAD // CHUNK
ND = 10240
RPT = ND // NS
NBUF = 4

_mesh = plsc.VectorSubcoreMesh(
    core_axis_name="c", subcore_axis_name="s", num_cores=NC, num_subcores=NS
)


def _prep_body(src_h, dst_h, ew_h, cur_h, z1_h, degp, cntp, mskp,
               sbuf, dbuf, ebuf, cbuf, obuf, hbuf, accd, accc, accm):
    c = lax.axis_index("c")
    s = lax.axis_index("s")
    wid = c * NS + s
    base = wid * KCH
    pltpu.sync_copy(src_h.at[pl.ds(base, KCH)], sbuf)
    pltpu.sync_copy(dst_h.at[pl.ds(base, KCH)], dbuf)
    pltpu.sync_copy(ew_h.at[pl.ds(base, KCH)], ebuf)
    pltpu.sync_copy(cur_h, cbuf)
    off = s * RPT
    pltpu.sync_copy(z1_h.at[pl.ds(off, RPT)], accd.at[pl.ds(off, RPT)])
    pltpu.sync_copy(z1_h.at[pl.ds(off, RPT)], accc.at[pl.ds(off, RPT)])
    pltpu.sync_copy(z1_h.at[pl.ds(off, RPT)], accm.at[pl.ds(off, RPT)])

    def fill_ones(k, carry):
        obuf[pl.ds(k * 16, 16)] = jnp.full((16,), 1.0, jnp.float32)
        return carry

    lax.fori_loop(0, CHUNK // 16, fill_ones, 0)
    plsc.subcore_barrier()
    cv = cbuf[...]

    def chunk(j, carry):
        def hk(k, inner):
            s16 = sbuf[j, pl.ds(k * 16, 16)]
            hbuf[pl.ds(k * 16, 16)] = jnp.where(
                s16 == cv, jnp.full((16,), 1.0, jnp.float32),
                jnp.full((16,), 0.0, jnp.float32))
            return inner

        lax.fori_loop(0, CHUNK // 16, hk, 0)
        pltpu.sync_copy(ebuf.at[j], accd.at[dbuf.at[j]], add=True)
        pltpu.sync_copy(obuf, accc.at[dbuf.at[j]], add=True)
        pltpu.sync_copy(hbuf, accm.at[dbuf.at[j]], add=True)
        return carry

    lax.fori_loop(0, KCH, chunk, 0)
    plsc.subcore_barrier()
    pltpu.sync_copy(accd.at[pl.ds(off, RPT)], degp.at[c, pl.ds(off, RPT)])
    pltpu.sync_copy(accc.at[pl.ds(off, RPT)], cntp.at[c, pl.ds(off, RPT)])
    pltpu.sync_copy(accm.at[pl.ds(off, RPT)], mskp.at[c, pl.ds(off, RPT)])


_prep = pl.kernel(
    _prep_body,
    out_type=(
        jax.ShapeDtypeStruct((NC, ND), jnp.float32),
        jax.ShapeDtypeStruct((NC, ND), jnp.float32),
        jax.ShapeDtypeStruct((NC, ND), jnp.float32),
    ),
    mesh=_mesh,
    scratch_types=[
        pltpu.VMEM((KCH, CHUNK), jnp.int32),
        pltpu.VMEM((KCH, CHUNK), jnp.int32),
        pltpu.VMEM((KCH, CHUNK), jnp.float32),
        pltpu.VMEM((16,), jnp.int32),
        pltpu.VMEM((CHUNK,), jnp.float32),
        pltpu.VMEM((CHUNK,), jnp.float32),
        pltpu.VMEM_SHARED((ND,), jnp.float32),
        pltpu.VMEM_SHARED((ND,), jnp.float32),
        pltpu.VMEM_SHARED((ND,), jnp.float32),
    ],
)


def _agg_body(weighted, src_h, dst_h, ew_h, y_h, z2_h, outp,
              sbuf, dbuf, ebuf, rows, acc, g0, g1, g2, g3):
    gs = (g0, g1, g2, g3)
    c = lax.axis_index("c")
    s = lax.axis_index("s")
    wid = c * NS + s
    base = wid * KCH
    pltpu.sync_copy(src_h.at[pl.ds(base, KCH)], sbuf)
    pltpu.sync_copy(dst_h.at[pl.ds(base, KCH)], dbuf)
    if weighted:
        pltpu.sync_copy(ew_h.at[pl.ds(base, KCH)], ebuf)
    off = s * RPT
    pltpu.sync_copy(z2_h.at[pl.ds(off, RPT)], acc.at[pl.ds(off, RPT)])
    plsc.subcore_barrier()
    for b in range(NBUF):
        pltpu.async_copy(y_h.at[sbuf.at[b]], rows.at[b], gs[b])

    def outer(t, carry):
        for b in range(NBUF):
            j = t * NBUF + b
            pltpu.make_async_copy(y_h.at[sbuf.at[b]], rows.at[b], gs[b]).wait()
            if weighted:
                def esc(k, inner):
                    wv = ebuf[j, pl.ds(k * 16, 16)]
                    for m in range(16):
                        i = k * 16 + m
                        w = wv[m]
                        rows[b, i, pl.ds(0, 16)] = rows[b, i, pl.ds(0, 16)] * w
                        rows[b, i, pl.ds(16, 16)] = (
                            rows[b, i, pl.ds(16, 16)] * w)
                    return inner

                lax.fori_loop(0, CHUNK // 16, esc, 0)
            pltpu.sync_copy(rows.at[b], acc.at[dbuf.at[j]], add=True)

            @pl.when(j + NBUF < KCH)
            def _():
                pltpu.async_copy(y_h.at[sbuf.at[j + NBUF]], rows.at[b], gs[b])

        return carry

    lax.fori_loop(0, KCH // NBUF, outer, 0)
    plsc.subcore_barrier()
    pltpu.sync_copy(acc.at[pl.ds(off, RPT)], outp.at[c, pl.ds(off, RPT)])


def _make_agg(weighted):
    return pl.kernel(
        functools.partial(_agg_body, weighted),
        out_type=jax.ShapeDtypeStruct((NC, ND, H), jnp.float32),
        mesh=_mesh,
        scratch_types=[
            pltpu.VMEM((KCH, CHUNK), jnp.int32),
            pltpu.VMEM((KCH, CHUNK), jnp.int32),
            pltpu.VMEM((KCH, CHUNK), jnp.float32),
            pltpu.VMEM((NBUF, CHUNK, H), jnp.float32),
            pltpu.VMEM_SHARED((ND, H), jnp.float32),
            pltpu.SemaphoreType.DMA,
            pltpu.SemaphoreType.DMA,
            pltpu.SemaphoreType.DMA,
            pltpu.SemaphoreType.DMA,
        ],
        compiler_params=pltpu.CompilerParams(use_tc_tiling_on_sc=False),
    )


_agg_w = _make_agg(True)
_agg_u = _make_agg(False)

RB = 256
GRID = ND // RB
RH = 40


def _k1_body(x_ref, w_ref, p0_ref, p1_ref, o_ref):
    dinv = lax.rsqrt(p0_ref[...] + p1_ref[...] + 1.0)
    o_ref[...] = jnp.dot(
        x_ref[...], w_ref[...], preferred_element_type=jnp.float32) * dinv


def _mid_body(z0_ref, z1_ref, y_ref, pp0_ref, pp1_ref, nn0_ref, nn1_ref,
              b_ref, w_ref, o_ref):
    dinvp = lax.rsqrt(pp0_ref[...] + pp1_ref[...] + 1.0)
    h = jnp.maximum(
        (z0_ref[...] + z1_ref[...] + y_ref[...]) * dinvp + b_ref[...], 0.0)
    dinvn = lax.rsqrt(nn0_ref[...] + nn1_ref[...] + 1.0)
    o_ref[...] = jnp.dot(
        h, w_ref[...], preferred_element_type=jnp.float32) * dinvn


def _head_body(z0_ref, z1_ref, y_ref, cp0_ref, cp1_ref, b5_ref,
               m0_ref, m1_ref, wp_ref, bp_ref, wv_ref, bv_ref,
               p_ref, v_ref):
    dinv2 = lax.rsqrt(cp0_ref[...] + cp1_ref[...] + 1.0)
    h = jnp.maximum(
        (z0_ref[...] + z1_ref[...] + y_ref[...]) * dinv2 + b5_ref[...], 0.0)
    msum = m0_ref[...] + m1_ref[...]
    maskf = jnp.where(msum > 0.0, 1.0, 0.0)
    anyrow = jnp.any(msum > 0.0)

    @pl.when(anyrow)
    def _():
        zp = jnp.dot(h, wp_ref[...], preferred_element_type=jnp.float32)
        zp = zp + bp_ref[...]
        zmax = jnp.max(zp, axis=1, keepdims=True)
        ez = jnp.exp(zp - zmax)
        ssum = jnp.sum(ez, axis=1, keepdims=True)
        p_ref[...] = ez * (maskf / ssum)
        zv = jnp.dot(h, wv_ref[...], preferred_element_type=jnp.float32)
        v_ref[...] = (zv + bv_ref[...]) * maskf

    @pl.when(jnp.logical_not(anyrow))
    def _():
        p_ref[...] = jnp.zeros((RH, O), jnp.float32)
        v_ref[...] = jnp.zeros((RH, O), jnp.float32)


def _row_spec(rows, cols):
    return pl.BlockSpec((rows, cols), lambda i: (i, 0))


def _const_spec(rows, cols):
    return pl.BlockSpec((rows, cols), lambda i: (0, 0))


def kernel(x, edge_index, edge_weight, current_node,
           W1, b1, W2, b2, W3, b3, W4, b4, W5, b5, Wp, bp, Wv, bv):
    src = edge_index[0]
    dst = edge_index[1]
    pad = EPAD - E
    srcp = jnp.concatenate(
        [src, jnp.full((pad,), N, jnp.int32)]).reshape(ECH, CHUNK)
    dstp = jnp.concatenate(
        [dst, jnp.full((pad,), N, jnp.int32)]).reshape(ECH, CHUNK)
    ewp = jnp.concatenate(
        [edge_weight, jnp.zeros((pad,), jnp.float32)]).reshape(ECH, CHUNK)
    cur16 = jnp.full((16,), current_node, jnp.int32)
    z1d = jnp.zeros((ND,), jnp.float32)
    z2d = jnp.zeros((ND, H), jnp.float32)
    x_p = jnp.pad(x, ((0, ND - N), (0, 0)))

    degp, cntp, mskp = _prep(srcp, dstp, ewp, cur16, z1d)
    d10 = degp[0].reshape(ND, 1)
    d11 = degp[1].reshape(ND, 1)
    c0 = cntp[0].reshape(ND, 1)
    c1 = cntp[1].reshape(ND, 1)
    m0 = mskp[0].reshape(ND, 1)
    m1 = mskp[1].reshape(ND, 1)

    y1 = pl.pallas_call(
        _k1_body,
        grid=(GRID,),
        in_specs=[
            _row_spec(RB, D_IN),
            _const_spec(D_IN, H),
            _row_spec(RB, 1),
            _row_spec(RB, 1),
        ],
        out_specs=_row_spec(RB, H),
        out_shape=jax.ShapeDtypeStruct((ND, H), jnp.float32),
    )(x_p, W1, d10, d11)

    def mid(zp, y_prev, pp0, pp1, b_prev, w_next):
        z0 = zp[0]
        z1 = zp[1]
        return pl.pallas_call(
            _mid_body,
            grid=(GRID,),
            in_specs=[
                _row_spec(RB, H),
                _row_spec(RB, H),
                _row_spec(RB, H),
                _row_spec(RB, 1),
                _row_spec(RB, 1),
                _row_spec(RB, 1),
                _row_spec(RB, 1),
                _const_spec(1, H),
                _const_spec(H, H),
            ],
            out_specs=_row_spec(RB, H),
            out_shape=jax.ShapeDtypeStruct((ND, H), jnp.float32),
        )(z0, z1, y_prev, pp0, pp1, c0, c1, b_prev.reshape(1, H), w_next)

    zp1 = _agg_w(srcp, dstp, ewp, y1, z2d)
    y2 = mid(zp1, y1, d10, d11, b1, W2)
    zp2 = _agg_u(srcp, dstp, ewp, y2, z2d)
    y3 = mid(zp2, y2, c0, c1, b2, W3)
    zp3 = _agg_u(srcp, dstp, ewp, y3, z2d)
    y4 = mid(zp3, y3, c0, c1, b3, W4)
    zp4 = _agg_u(srcp, dstp, ewp, y4, z2d)
    y5 = mid(zp4, y4, c0, c1, b4, W5)
    zp5 = _agg_u(srcp, dstp, ewp, y5, z2d)

    p, v = pl.pallas_call(
        _head_body,
        grid=(N // RH,),
        in_specs=[
            _row_spec(RH, H),
            _row_spec(RH, H),
            _row_spec(RH, H),
            _row_spec(RH, 1),
            _row_spec(RH, 1),
            _const_spec(1, H),
            _row_spec(RH, 1),
            _row_spec(RH, 1),
            _const_spec(H, O),
            _const_spec(1, O),
            _const_spec(H, O),
            _const_spec(1, O),
        ],
        out_specs=[_row_spec(RH, O), _row_spec(RH, O)],
        out_shape=[
            jax.ShapeDtypeStruct((N, O), jnp.float32),
            jax.ShapeDtypeStruct((N, O), jnp.float32),
        ],
    )(zp5[0], zp5[1], y5, c0, c1, b5.reshape(1, H), m0, m1,
      Wp, bp.reshape(1, O), Wv, bv.reshape(1, O))
    return (p, v)


# speedup vs baseline: 14.9359x; 1.0590x over previous
import functools

import jax
import jax.numpy as jnp
from jax import lax
from jax.experimental import pallas as pl
from jax.experimental.pallas import tpu as pltpu
from jax.experimental.pallas import tpu_sc as plsc

N = 10000
E = 160000
D_IN = 256
H = 32
O = 10000

NC = 2
NS = 16
NW = NC * NS
CHUNK = 128
KCH = 40
EPW = KCH * CHUNK
EPAD = EPW * NW
ECH = EPAD // CHUNK
ND = 10240
RPT = ND // NS
NBUF = 4

_mesh = plsc.VectorSubcoreMesh(
    core_axis_name="c", subcore_axis_name="s", num_cores=NC, num_subcores=NS
)


def _prep_body(src_h, dst_h, ew_h, cur_h, z1_h, degp, cntp, mskp,
               sbuf, dbuf, ebuf, cbuf, obuf, hbuf, accd, accc, accm):
    c = lax.axis_index("c")
    s = lax.axis_index("s")
    wid = c * NS + s
    base = wid * KCH
    pltpu.sync_copy(src_h.at[pl.ds(base, KCH)], sbuf)
    pltpu.sync_copy(dst_h.at[pl.ds(base, KCH)], dbuf)
    pltpu.sync_copy(ew_h.at[pl.ds(base, KCH)], ebuf)
    pltpu.sync_copy(cur_h, cbuf)
    off = s * RPT
    pltpu.sync_copy(z1_h.at[pl.ds(off, RPT)], accd.at[pl.ds(off, RPT)])
    pltpu.sync_copy(z1_h.at[pl.ds(off, RPT)], accc.at[pl.ds(off, RPT)])
    pltpu.sync_copy(z1_h.at[pl.ds(off, RPT)], accm.at[pl.ds(off, RPT)])

    def fill_ones(k, carry):
        obuf[pl.ds(k * 16, 16)] = jnp.full((16,), 1.0, jnp.float32)
        return carry

    lax.fori_loop(0, CHUNK // 16, fill_ones, 0)
    plsc.subcore_barrier()
    cv = cbuf[...]

    def chunk(j, carry):
        def hk(k, inner):
            s16 = sbuf[j, pl.ds(k * 16, 16)]
            hbuf[pl.ds(k * 16, 16)] = jnp.where(
                s16 == cv, jnp.full((16,), 1.0, jnp.float32),
                jnp.full((16,), 0.0, jnp.float32))
            return inner

        lax.fori_loop(0, CHUNK // 16, hk, 0)
        pltpu.sync_copy(ebuf.at[j], accd.at[dbuf.at[j]], add=True)
        pltpu.sync_copy(obuf, accc.at[dbuf.at[j]], add=True)
        pltpu.sync_copy(hbuf, accm.at[dbuf.at[j]], add=True)
        return carry

    lax.fori_loop(0, KCH, chunk, 0)
    plsc.subcore_barrier()
    pltpu.sync_copy(accd.at[pl.ds(off, RPT)], degp.at[c, pl.ds(off, RPT)])
    pltpu.sync_copy(accc.at[pl.ds(off, RPT)], cntp.at[c, pl.ds(off, RPT)])
    pltpu.sync_copy(accm.at[pl.ds(off, RPT)], mskp.at[c, pl.ds(off, RPT)])


_prep = pl.kernel(
    _prep_body,
    out_type=(
        jax.ShapeDtypeStruct((NC, ND), jnp.float32),
        jax.ShapeDtypeStruct((NC, ND), jnp.float32),
        jax.ShapeDtypeStruct((NC, ND), jnp.float32),
    ),
    mesh=_mesh,
    scratch_types=[
        pltpu.VMEM((KCH, CHUNK), jnp.int32),
        pltpu.VMEM((KCH, CHUNK), jnp.int32),
        pltpu.VMEM((KCH, CHUNK), jnp.float32),
        pltpu.VMEM((16,), jnp.int32),
        pltpu.VMEM((CHUNK,), jnp.float32),
        pltpu.VMEM((CHUNK,), jnp.float32),
        pltpu.VMEM_SHARED((ND,), jnp.float32),
        pltpu.VMEM_SHARED((ND,), jnp.float32),
        pltpu.VMEM_SHARED((ND,), jnp.float32),
    ],
)


def _agg_body(weighted, src_h, dst_h, ew_h, y_h, z2_h, outp,
              sbuf, dbuf, ebuf, rows, acc, g0, g1, g2, g3):
    gs = (g0, g1, g2, g3)
    c = lax.axis_index("c")
    s = lax.axis_index("s")
    wid = c * NS + s
    base = wid * KCH
    pltpu.sync_copy(src_h.at[pl.ds(base, KCH)], sbuf)
    pltpu.sync_copy(dst_h.at[pl.ds(base, KCH)], dbuf)
    if weighted:
        pltpu.sync_copy(ew_h.at[pl.ds(base, KCH)], ebuf)
    off = s * RPT
    pltpu.sync_copy(z2_h.at[pl.ds(off, RPT)], acc.at[pl.ds(off, RPT)])
    plsc.subcore_barrier()
    for b in range(NBUF):
        pltpu.async_copy(y_h.at[sbuf.at[b]], rows.at[b], gs[b])

    def outer(t, carry):
        for b in range(NBUF):
            j = t * NBUF + b
            pltpu.make_async_copy(y_h.at[sbuf.at[b]], rows.at[b], gs[b]).wait()
            if weighted:
                def esc(k, inner):
                    wv = ebuf[j, pl.ds(k * 16, 16)]
                    for m in range(16):
                        i = k * 16 + m
                        w = wv[m]
                        rows[b, i, pl.ds(0, 16)] = rows[b, i, pl.ds(0, 16)] * w
                        rows[b, i, pl.ds(16, 16)] = (
                            rows[b, i, pl.ds(16, 16)] * w)
                    return inner

                lax.fori_loop(0, CHUNK // 16, esc, 0)
            pltpu.sync_copy(rows.at[b], acc.at[dbuf.at[j]], add=True)

            @pl.when(j + NBUF < KCH)
            def _():
                pltpu.async_copy(y_h.at[sbuf.at[j + NBUF]], rows.at[b], gs[b])

        return carry

    lax.fori_loop(0, KCH // NBUF, outer, 0)
    plsc.subcore_barrier()
    pltpu.sync_copy(acc.at[pl.ds(off, RPT)], outp.at[c, pl.ds(off, RPT)])


def _make_agg(weighted):
    return pl.kernel(
        functools.partial(_agg_body, weighted),
        out_type=jax.ShapeDtypeStruct((NC, ND, H), jnp.float32),
        mesh=_mesh,
        scratch_types=[
            pltpu.VMEM((KCH, CHUNK), jnp.int32),
            pltpu.VMEM((KCH, CHUNK), jnp.int32),
            pltpu.VMEM((KCH, CHUNK), jnp.float32),
            pltpu.VMEM((NBUF, CHUNK, H), jnp.float32),
            pltpu.VMEM_SHARED((ND, H), jnp.float32),
            pltpu.SemaphoreType.DMA,
            pltpu.SemaphoreType.DMA,
            pltpu.SemaphoreType.DMA,
            pltpu.SemaphoreType.DMA,
        ],
        compiler_params=pltpu.CompilerParams(use_tc_tiling_on_sc=False),
    )


_agg_w = _make_agg(True)
_agg_u = _make_agg(False)

RB = 256
GRID = ND // RB
RH = 80


def _k1_body(x_ref, w_ref, p0_ref, p1_ref, o_ref):
    dinv = lax.rsqrt(p0_ref[...] + p1_ref[...] + 1.0)
    o_ref[...] = jnp.dot(
        x_ref[...], w_ref[...], preferred_element_type=jnp.float32) * dinv


def _mid_body(z0_ref, z1_ref, y_ref, pp0_ref, pp1_ref, nn0_ref, nn1_ref,
              b_ref, w_ref, o_ref):
    dinvp = lax.rsqrt(pp0_ref[...] + pp1_ref[...] + 1.0)
    h = jnp.maximum(
        (z0_ref[...] + z1_ref[...] + y_ref[...]) * dinvp + b_ref[...], 0.0)
    dinvn = lax.rsqrt(nn0_ref[...] + nn1_ref[...] + 1.0)
    o_ref[...] = jnp.dot(
        h, w_ref[...], preferred_element_type=jnp.float32) * dinvn


def _head_body(z0_ref, z1_ref, y_ref, cp0_ref, cp1_ref, b5_ref,
               m0_ref, m1_ref, wp_ref, bp_ref, wv_ref, bv_ref,
               p_ref, v_ref):
    dinv2 = lax.rsqrt(cp0_ref[...] + cp1_ref[...] + 1.0)
    h = jnp.maximum(
        (z0_ref[...] + z1_ref[...] + y_ref[...]) * dinv2 + b5_ref[...], 0.0)
    msum = m0_ref[...] + m1_ref[...]
    maskf = jnp.where(msum > 0.0, 1.0, 0.0)
    anyrow = jnp.any(msum > 0.0)

    @pl.when(anyrow)
    def _():
        zp = jnp.dot(h, wp_ref[...], preferred_element_type=jnp.float32)
        zp = zp + bp_ref[...]
        zmax = jnp.max(zp, axis=1, keepdims=True)
        ez = jnp.exp(zp - zmax)
        ssum = jnp.sum(ez, axis=1, keepdims=True)
        p_ref[...] = ez * (maskf / ssum)
        zv = jnp.dot(h, wv_ref[...], preferred_element_type=jnp.float32)
        v_ref[...] = (zv + bv_ref[...]) * maskf

    @pl.when(jnp.logical_not(anyrow))
    def _():
        p_ref[...] = jnp.zeros((RH, O), jnp.float32)
        v_ref[...] = jnp.zeros((RH, O), jnp.float32)


def _row_spec(rows, cols):
    return pl.BlockSpec((rows, cols), lambda i: (i, 0))


def _const_spec(rows, cols):
    return pl.BlockSpec((rows, cols), lambda i: (0, 0))


def kernel(x, edge_index, edge_weight, current_node,
           W1, b1, W2, b2, W3, b3, W4, b4, W5, b5, Wp, bp, Wv, bv):
    src = edge_index[0]
    dst = edge_index[1]
    pad = EPAD - E
    srcp = jnp.concatenate(
        [src, jnp.full((pad,), N, jnp.int32)]).reshape(ECH, CHUNK)
    dstp = jnp.concatenate(
        [dst, jnp.full((pad,), N, jnp.int32)]).reshape(ECH, CHUNK)
    ewp = jnp.concatenate(
        [edge_weight, jnp.zeros((pad,), jnp.float32)]).reshape(ECH, CHUNK)
    cur16 = jnp.full((16,), current_node, jnp.int32)
    z1d = jnp.zeros((ND,), jnp.float32)
    z2d = jnp.zeros((ND, H), jnp.float32)
    x_p = jnp.pad(x, ((0, ND - N), (0, 0)))

    degp, cntp, mskp = _prep(srcp, dstp, ewp, cur16, z1d)
    d10 = degp[0].reshape(ND, 1)
    d11 = degp[1].reshape(ND, 1)
    c0 = cntp[0].reshape(ND, 1)
    c1 = cntp[1].reshape(ND, 1)
    m0 = mskp[0].reshape(ND, 1)
    m1 = mskp[1].reshape(ND, 1)

    y1 = pl.pallas_call(
        _k1_body,
        grid=(GRID,),
        in_specs=[
            _row_spec(RB, D_IN),
            _const_spec(D_IN, H),
            _row_spec(RB, 1),
            _row_spec(RB, 1),
        ],
        out_specs=_row_spec(RB, H),
        out_shape=jax.ShapeDtypeStruct((ND, H), jnp.float32),
    )(x_p, W1, d10, d11)

    def mid(zp, y_prev, pp0, pp1, b_prev, w_next):
        z0 = zp[0]
        z1 = zp[1]
        return pl.pallas_call(
            _mid_body,
            grid=(GRID,),
            in_specs=[
                _row_spec(RB, H),
                _row_spec(RB, H),
                _row_spec(RB, H),
                _row_spec(RB, 1),
                _row_spec(RB, 1),
                _row_spec(RB, 1),
                _row_spec(RB, 1),
                _const_spec(1, H),
                _const_spec(H, H),
            ],
            out_specs=_row_spec(RB, H),
            out_shape=jax.ShapeDtypeStruct((ND, H), jnp.float32),
        )(z0, z1, y_prev, pp0, pp1, c0, c1, b_prev.reshape(1, H), w_next)

    zp1 = _agg_w(srcp, dstp, ewp, y1, z2d)
    y2 = mid(zp1, y1, d10, d11, b1, W2)
    zp2 = _agg_u(srcp, dstp, ewp, y2, z2d)
    y3 = mid(zp2, y2, c0, c1, b2, W3)
    zp3 = _agg_u(srcp, dstp, ewp, y3, z2d)
    y4 = mid(zp3, y3, c0, c1, b3, W4)
    zp4 = _agg_u(srcp, dstp, ewp, y4, z2d)
    y5 = mid(zp4, y4, c0, c1, b4, W5)
    zp5 = _agg_u(srcp, dstp, ewp, y5, z2d)

    p, v = pl.pallas_call(
        _head_body,
        grid=(N // RH,),
        in_specs=[
            _row_spec(RH, H),
            _row_spec(RH, H),
            _row_spec(RH, H),
            _row_spec(RH, 1),
            _row_spec(RH, 1),
            _const_spec(1, H),
            _row_spec(RH, 1),
            _row_spec(RH, 1),
            _const_spec(H, O),
            _const_spec(1, O),
            _const_spec(H, O),
            _const_spec(1, O),
        ],
        out_specs=[_row_spec(RH, O), _row_spec(RH, O)],
        out_shape=[
            jax.ShapeDtypeStruct((N, O), jnp.float32),
            jax.ShapeDtypeStruct((N, O), jnp.float32),
        ],
    )(zp5[0], zp5[1], y5, c0, c1, b5.reshape(1, H), m0, m1,
      Wp, bp.reshape(1, O), Wv, bv.reshape(1, O))
    return (p, v)


# speedup vs baseline: 15.4017x; 1.0312x over previous
import functools

import jax
import jax.numpy as jnp
from jax import lax
from jax.experimental import pallas as pl
from jax.experimental.pallas import tpu as pltpu
from jax.experimental.pallas import tpu_sc as plsc

N = 10000
E = 160000
D_IN = 256
H = 32
O = 10000

NC = 2
NS = 16
NW = NC * NS
CHUNK = 128
KCH = 40
EPW = KCH * CHUNK
EPAD = EPW * NW
ECH = EPAD // CHUNK
ND = 10240
RPT = ND // NS
NBUF = 8

_mesh = plsc.VectorSubcoreMesh(
    core_axis_name="c", subcore_axis_name="s", num_cores=NC, num_subcores=NS
)


def _prep_body(src_h, dst_h, ew_h, cur_h, z1_h, degp, cntp, mskp,
               sbuf, dbuf, ebuf, cbuf, obuf, hbuf, accd, accc, accm):
    c = lax.axis_index("c")
    s = lax.axis_index("s")
    wid = c * NS + s
    base = wid * KCH
    pltpu.sync_copy(src_h.at[pl.ds(base, KCH)], sbuf)
    pltpu.sync_copy(dst_h.at[pl.ds(base, KCH)], dbuf)
    pltpu.sync_copy(ew_h.at[pl.ds(base, KCH)], ebuf)
    pltpu.sync_copy(cur_h, cbuf)
    off = s * RPT
    pltpu.sync_copy(z1_h.at[pl.ds(off, RPT)], accd.at[pl.ds(off, RPT)])
    pltpu.sync_copy(z1_h.at[pl.ds(off, RPT)], accc.at[pl.ds(off, RPT)])
    pltpu.sync_copy(z1_h.at[pl.ds(off, RPT)], accm.at[pl.ds(off, RPT)])

    def fill_ones(k, carry):
        obuf[pl.ds(k * 16, 16)] = jnp.full((16,), 1.0, jnp.float32)
        return carry

    lax.fori_loop(0, CHUNK // 16, fill_ones, 0)
    plsc.subcore_barrier()
    cv = cbuf[...]

    def chunk(j, carry):
        def hk(k, inner):
            s16 = sbuf[j, pl.ds(k * 16, 16)]
            hbuf[pl.ds(k * 16, 16)] = jnp.where(
                s16 == cv, jnp.full((16,), 1.0, jnp.float32),
                jnp.full((16,), 0.0, jnp.float32))
            return inner

        lax.fori_loop(0, CHUNK // 16, hk, 0)
        pltpu.sync_copy(ebuf.at[j], accd.at[dbuf.at[j]], add=True)
        pltpu.sync_copy(obuf, accc.at[dbuf.at[j]], add=True)
        pltpu.sync_copy(hbuf, accm.at[dbuf.at[j]], add=True)
        return carry

    lax.fori_loop(0, KCH, chunk, 0)
    plsc.subcore_barrier()
    pltpu.sync_copy(accd.at[pl.ds(off, RPT)], degp.at[c, pl.ds(off, RPT)])
    pltpu.sync_copy(accc.at[pl.ds(off, RPT)], cntp.at[c, pl.ds(off, RPT)])
    pltpu.sync_copy(accm.at[pl.ds(off, RPT)], mskp.at[c, pl.ds(off, RPT)])


_prep = pl.kernel(
    _prep_body,
    out_type=(
        jax.ShapeDtypeStruct((NC, ND), jnp.float32),
        jax.ShapeDtypeStruct((NC, ND), jnp.float32),
        jax.ShapeDtypeStruct((NC, ND), jnp.float32),
    ),
    mesh=_mesh,
    scratch_types=[
        pltpu.VMEM((KCH, CHUNK), jnp.int32),
        pltpu.VMEM((KCH, CHUNK), jnp.int32),
        pltpu.VMEM((KCH, CHUNK), jnp.float32),
        pltpu.VMEM((16,), jnp.int32),
        pltpu.VMEM((CHUNK,), jnp.float32),
        pltpu.VMEM((CHUNK,), jnp.float32),
        pltpu.VMEM_SHARED((ND,), jnp.float32),
        pltpu.VMEM_SHARED((ND,), jnp.float32),
        pltpu.VMEM_SHARED((ND,), jnp.float32),
    ],
)


def _agg_body(weighted, src_h, dst_h, ew_h, y_h, z2_h, outp,
              sbuf, dbuf, ebuf, rows, acc, gsems, ssems):
    c = lax.axis_index("c")
    s = lax.axis_index("s")
    wid = c * NS + s
    base = wid * KCH
    pltpu.sync_copy(src_h.at[pl.ds(base, KCH)], sbuf)
    pltpu.sync_copy(dst_h.at[pl.ds(base, KCH)], dbuf)
    if weighted:
        pltpu.sync_copy(ew_h.at[pl.ds(base, KCH)], ebuf)
    off = s * RPT
    pltpu.sync_copy(z2_h.at[pl.ds(off, RPT)], acc.at[pl.ds(off, RPT)])
    plsc.subcore_barrier()
    for b in range(NBUF):
        pltpu.async_copy(y_h.at[sbuf.at[b]], rows.at[b], gsems.at[b])

    for i in range(KCH):
        b = i % NBUF
        pltpu.make_async_copy(
            y_h.at[sbuf.at[i]], rows.at[b], gsems.at[b]).wait()
        if weighted:
            def esc(k, inner, i=i, b=b):
                wv = ebuf[i, pl.ds(k * 16, 16)]
                for m in range(16):
                    r = k * 16 + m
                    w = wv[m]
                    rows[b, r, pl.ds(0, 16)] = rows[b, r, pl.ds(0, 16)] * w
                    rows[b, r, pl.ds(16, 16)] = rows[b, r, pl.ds(16, 16)] * w
                return inner

            lax.fori_loop(0, CHUNK // 16, esc, 0)
        pltpu.async_copy(rows.at[b], acc.at[dbuf.at[i]], ssems.at[b], add=True)
        ir = i + NBUF // 2
        if NBUF <= ir < KCH:
            br = ir % NBUF
            pltpu.make_async_copy(
                rows.at[br], acc.at[dbuf.at[ir - NBUF]], ssems.at[br]).wait()
            pltpu.async_copy(y_h.at[sbuf.at[ir]], rows.at[br], gsems.at[br])

    for b in range(NBUF):
        i_last = KCH - NBUF + b
        pltpu.make_async_copy(
            rows.at[b], acc.at[dbuf.at[i_last]], ssems.at[b]).wait()
    plsc.subcore_barrier()
    pltpu.sync_copy(acc.at[pl.ds(off, RPT)], outp.at[c, pl.ds(off, RPT)])


def _make_agg(weighted):
    return pl.kernel(
        functools.partial(_agg_body, weighted),
        out_type=jax.ShapeDtypeStruct((NC, ND, H), jnp.float32),
        mesh=_mesh,
        scratch_types=[
            pltpu.VMEM((KCH, CHUNK), jnp.int32),
            pltpu.VMEM((KCH, CHUNK), jnp.int32),
            pltpu.VMEM((KCH, CHUNK), jnp.float32),
            pltpu.VMEM((NBUF, CHUNK, H), jnp.float32),
            pltpu.VMEM_SHARED((ND, H), jnp.float32),
            pltpu.SemaphoreType.DMA((NBUF,)),
            pltpu.SemaphoreType.DMA((NBUF,)),
        ],
        compiler_params=pltpu.CompilerParams(use_tc_tiling_on_sc=False),
    )


_agg_w = _make_agg(True)
_agg_u = _make_agg(False)

RB = 256
GRID = ND // RB
RH = 80


def _k1_body(x_ref, w_ref, p0_ref, p1_ref, o_ref):
    dinv = lax.rsqrt(p0_ref[...] + p1_ref[...] + 1.0)
    o_ref[...] = jnp.dot(
        x_ref[...], w_ref[...], preferred_element_type=jnp.float32) * dinv


def _mid_body(z0_ref, z1_ref, y_ref, pp0_ref, pp1_ref, nn0_ref, nn1_ref,
              b_ref, w_ref, o_ref):
    dinvp = lax.rsqrt(pp0_ref[...] + pp1_ref[...] + 1.0)
    h = jnp.maximum(
        (z0_ref[...] + z1_ref[...] + y_ref[...]) * dinvp + b_ref[...], 0.0)
    dinvn = lax.rsqrt(nn0_ref[...] + nn1_ref[...] + 1.0)
    o_ref[...] = jnp.dot(
        h, w_ref[...], preferred_element_type=jnp.float32) * dinvn


def _head_body(z0_ref, z1_ref, y_ref, cp0_ref, cp1_ref, b5_ref,
               m0_ref, m1_ref, wp_ref, bp_ref, wv_ref, bv_ref,
               p_ref, v_ref):
    dinv2 = lax.rsqrt(cp0_ref[...] + cp1_ref[...] + 1.0)
    h = jnp.maximum(
        (z0_ref[...] + z1_ref[...] + y_ref[...]) * dinv2 + b5_ref[...], 0.0)
    msum = m0_ref[...] + m1_ref[...]
    maskf = jnp.where(msum > 0.0, 1.0, 0.0)
    anyrow = jnp.any(msum > 0.0)

    @pl.when(anyrow)
    def _():
        zp = jnp.dot(h, wp_ref[...], preferred_element_type=jnp.float32)
        zp = zp + bp_ref[...]
        zmax = jnp.max(zp, axis=1, keepdims=True)
        ez = jnp.exp(zp - zmax)
        ssum = jnp.sum(ez, axis=1, keepdims=True)
        p_ref[...] = ez * (maskf / ssum)
        zv = jnp.dot(h, wv_ref[...], preferred_element_type=jnp.float32)
        v_ref[...] = (zv + bv_ref[...]) * maskf

    @pl.when(jnp.logical_not(anyrow))
    def _():
        p_ref[...] = jnp.zeros((RH, O), jnp.float32)
        v_ref[...] = jnp.zeros((RH, O), jnp.float32)


def _row_spec(rows, cols):
    return pl.BlockSpec((rows, cols), lambda i: (i, 0))


def _const_spec(rows, cols):
    return pl.BlockSpec((rows, cols), lambda i: (0, 0))


def kernel(x, edge_index, edge_weight, current_node,
           W1, b1, W2, b2, W3, b3, W4, b4, W5, b5, Wp, bp, Wv, bv):
    src = edge_index[0]
    dst = edge_index[1]
    pad = EPAD - E
    srcp = jnp.concatenate(
        [src, jnp.full((pad,), N, jnp.int32)]).reshape(ECH, CHUNK)
    dstp = jnp.concatenate(
        [dst, jnp.full((pad,), N, jnp.int32)]).reshape(ECH, CHUNK)
    ewp = jnp.concatenate(
        [edge_weight, jnp.zeros((pad,), jnp.float32)]).reshape(ECH, CHUNK)
    cur16 = jnp.full((16,), current_node, jnp.int32)
    z1d = jnp.zeros((ND,), jnp.float32)
    z2d = jnp.zeros((ND, H), jnp.float32)
    x_p = jnp.pad(x, ((0, ND - N), (0, 0)))

    degp, cntp, mskp = _prep(srcp, dstp, ewp, cur16, z1d)
    d10 = degp[0].reshape(ND, 1)
    d11 = degp[1].reshape(ND, 1)
    c0 = cntp[0].reshape(ND, 1)
    c1 = cntp[1].reshape(ND, 1)
    m0 = mskp[0].reshape(ND, 1)
    m1 = mskp[1].reshape(ND, 1)

    y1 = pl.pallas_call(
        _k1_body,
        grid=(GRID,),
        in_specs=[
            _row_spec(RB, D_IN),
            _const_spec(D_IN, H),
            _row_spec(RB, 1),
            _row_spec(RB, 1),
        ],
        out_specs=_row_spec(RB, H),
        out_shape=jax.ShapeDtypeStruct((ND, H), jnp.float32),
    )(x_p, W1, d10, d11)

    def mid(zp, y_prev, pp0, pp1, b_prev, w_next):
        z0 = zp[0]
        z1 = zp[1]
        return pl.pallas_call(
            _mid_body,
            grid=(GRID,),
            in_specs=[
                _row_spec(RB, H),
                _row_spec(RB, H),
                _row_spec(RB, H),
                _row_spec(RB, 1),
                _row_spec(RB, 1),
                _row_spec(RB, 1),
                _row_spec(RB, 1),
                _const_spec(1, H),
                _const_spec(H, H),
            ],
            out_specs=_row_spec(RB, H),
            out_shape=jax.ShapeDtypeStruct((ND, H), jnp.float32),
        )(z0, z1, y_prev, pp0, pp1, c0, c1, b_prev.reshape(1, H), w_next)

    zp1 = _agg_w(srcp, dstp, ewp, y1, z2d)
    y2 = mid(zp1, y1, d10, d11, b1, W2)
    zp2 = _agg_u(srcp, dstp, ewp, y2, z2d)
    y3 = mid(zp2, y2, c0, c1, b2, W3)
    zp3 = _agg_u(srcp, dstp, ewp, y3, z2d)
    y4 = mid(zp3, y3, c0, c1, b3, W4)
    zp4 = _agg_u(srcp, dstp, ewp, y4, z2d)
    y5 = mid(zp4, y4, c0, c1, b4, W5)
    zp5 = _agg_u(srcp, dstp, ewp, y5, z2d)

    p, v = pl.pallas_call(
        _head_body,
        grid=(N // RH,),
        in_specs=[
            _row_spec(RH, H),
            _row_spec(RH, H),
            _row_spec(RH, H),
            _row_spec(RH, 1),
            _row_spec(RH, 1),
            _const_spec(1, H),
            _row_spec(RH, 1),
            _row_spec(RH, 1),
            _const_spec(H, O),
            _const_spec(1, O),
            _const_spec(H, O),
            _const_spec(1, O),
        ],
        out_specs=[_row_spec(RH, O), _row_spec(RH, O)],
        out_shape=[
            jax.ShapeDtypeStruct((N, O), jnp.float32),
            jax.ShapeDtypeStruct((N, O), jnp.float32),
        ],
    )(zp5[0], zp5[1], y5, c0, c1, b5.reshape(1, H), m0, m1,
      Wp, bp.reshape(1, O), Wv, bv.reshape(1, O))
    return (p, v)


# speedup vs baseline: 21.5221x; 1.3974x over previous
import functools

import jax
import jax.numpy as jnp
from jax import lax
from jax.experimental import pallas as pl
from jax.experimental.pallas import tpu as pltpu
from jax.experimental.pallas import tpu_sc as plsc

N = 10000
E = 160000
D_IN = 256
H = 32
O = 10000

NC = 2
NS = 16
NW = NC * NS
CHUNK = 128
KCH = 40
EPW = KCH * CHUNK
EPAD = EPW * NW
ECH = EPAD // CHUNK
ND = 10240
RPT = ND // NS
NBUF = 8

_mesh = plsc.VectorSubcoreMesh(
    core_axis_name="c", subcore_axis_name="s", num_cores=NC, num_subcores=NS
)


def _prep_body(src_h, dst_h, ew_h, cur_h, z1_h, degp, cntp, mskp,
               sbuf, dbuf, ebuf, cbuf, obuf, hbuf, accd, accc, accm):
    c = lax.axis_index("c")
    s = lax.axis_index("s")
    wid = c * NS + s
    base = wid * KCH
    pltpu.sync_copy(src_h.at[pl.ds(base, KCH)], sbuf)
    pltpu.sync_copy(dst_h.at[pl.ds(base, KCH)], dbuf)
    pltpu.sync_copy(ew_h.at[pl.ds(base, KCH)], ebuf)
    pltpu.sync_copy(cur_h, cbuf)
    off = s * RPT
    pltpu.sync_copy(z1_h.at[pl.ds(off, RPT)], accd.at[pl.ds(off, RPT)])
    pltpu.sync_copy(z1_h.at[pl.ds(off, RPT)], accc.at[pl.ds(off, RPT)])
    pltpu.sync_copy(z1_h.at[pl.ds(off, RPT)], accm.at[pl.ds(off, RPT)])

    def fill_ones(k, carry):
        obuf[pl.ds(k * 16, 16)] = jnp.full((16,), 1.0, jnp.float32)
        return carry

    lax.fori_loop(0, CHUNK // 16, fill_ones, 0)
    plsc.subcore_barrier()
    cv = cbuf[...]

    def chunk(j, carry):
        def hk(k, inner):
            s16 = sbuf[j, pl.ds(k * 16, 16)]
            hbuf[pl.ds(k * 16, 16)] = jnp.where(
                s16 == cv, jnp.full((16,), 1.0, jnp.float32),
                jnp.full((16,), 0.0, jnp.float32))
            return inner

        lax.fori_loop(0, CHUNK // 16, hk, 0)
        pltpu.sync_copy(ebuf.at[j], accd.at[dbuf.at[j]], add=True)
        pltpu.sync_copy(obuf, accc.at[dbuf.at[j]], add=True)
        pltpu.sync_copy(hbuf, accm.at[dbuf.at[j]], add=True)
        return carry

    lax.fori_loop(0, KCH, chunk, 0)
    plsc.subcore_barrier()
    pltpu.sync_copy(accd.at[pl.ds(off, RPT)], degp.at[c, pl.ds(off, RPT)])
    pltpu.sync_copy(accc.at[pl.ds(off, RPT)], cntp.at[c, pl.ds(off, RPT)])
    pltpu.sync_copy(accm.at[pl.ds(off, RPT)], mskp.at[c, pl.ds(off, RPT)])


_prep = pl.kernel(
    _prep_body,
    out_type=(
        jax.ShapeDtypeStruct((NC, ND), jnp.float32),
        jax.ShapeDtypeStruct((NC, ND), jnp.float32),
        jax.ShapeDtypeStruct((NC, ND), jnp.float32),
    ),
    mesh=_mesh,
    scratch_types=[
        pltpu.VMEM((KCH, CHUNK), jnp.int32),
        pltpu.VMEM((KCH, CHUNK), jnp.int32),
        pltpu.VMEM((KCH, CHUNK), jnp.float32),
        pltpu.VMEM((16,), jnp.int32),
        pltpu.VMEM((CHUNK,), jnp.float32),
        pltpu.VMEM((CHUNK,), jnp.float32),
        pltpu.VMEM_SHARED((ND,), jnp.float32),
        pltpu.VMEM_SHARED((ND,), jnp.float32),
        pltpu.VMEM_SHARED((ND,), jnp.float32),
    ],
)


def _agg_body(weighted, src_h, dst_h, ew_h, y_h, z2_h, outp,
              sbuf, dbuf, ebuf, rows, ystage, acc, gsems, ssems):
    c = lax.axis_index("c")
    s = lax.axis_index("s")
    wid = c * NS + s
    base = wid * KCH
    pltpu.sync_copy(src_h.at[pl.ds(base, KCH)], sbuf)
    pltpu.sync_copy(dst_h.at[pl.ds(base, KCH)], dbuf)
    if weighted:
        pltpu.sync_copy(ew_h.at[pl.ds(base, KCH)], ebuf)
    off = s * RPT
    pltpu.sync_copy(z2_h.at[pl.ds(off, RPT)], acc.at[pl.ds(off, RPT)])
    pltpu.sync_copy(y_h.at[pl.ds(off, RPT)], ystage.at[pl.ds(off, RPT)])
    plsc.subcore_barrier()
    for b in range(NBUF):
        pltpu.async_copy(ystage.at[sbuf.at[b]], rows.at[b], gsems.at[b])

    for i in range(KCH):
        b = i % NBUF
        pltpu.make_async_copy(
            ystage.at[sbuf.at[i]], rows.at[b], gsems.at[b]).wait()
        if weighted:
            def esc(k, inner, i=i, b=b):
                wv = ebuf[i, pl.ds(k * 16, 16)]
                for m in range(16):
                    r = k * 16 + m
                    w = wv[m]
                    rows[b, r, pl.ds(0, 16)] = rows[b, r, pl.ds(0, 16)] * w
                    rows[b, r, pl.ds(16, 16)] = rows[b, r, pl.ds(16, 16)] * w
                return inner

            lax.fori_loop(0, CHUNK // 16, esc, 0)
        pltpu.async_copy(rows.at[b], acc.at[dbuf.at[i]], ssems.at[b], add=True)
        ir = i + NBUF // 2
        if NBUF <= ir < KCH:
            br = ir % NBUF
            pltpu.make_async_copy(
                rows.at[br], acc.at[dbuf.at[ir - NBUF]], ssems.at[br]).wait()
            pltpu.async_copy(ystage.at[sbuf.at[ir]], rows.at[br], gsems.at[br])

    for b in range(NBUF):
        i_last = KCH - NBUF + b
        pltpu.make_async_copy(
            rows.at[b], acc.at[dbuf.at[i_last]], ssems.at[b]).wait()
    plsc.subcore_barrier()
    pltpu.sync_copy(acc.at[pl.ds(off, RPT)], outp.at[c, pl.ds(off, RPT)])


def _make_agg(weighted):
    return pl.kernel(
        functools.partial(_agg_body, weighted),
        out_type=jax.ShapeDtypeStruct((NC, ND, H), jnp.float32),
        mesh=_mesh,
        scratch_types=[
            pltpu.VMEM((KCH, CHUNK), jnp.int32),
            pltpu.VMEM((KCH, CHUNK), jnp.int32),
            pltpu.VMEM((KCH, CHUNK), jnp.float32),
            pltpu.VMEM((NBUF, CHUNK, H), jnp.float32),
            pltpu.VMEM_SHARED((ND, H), jnp.float32),
            pltpu.VMEM_SHARED((ND, H), jnp.float32),
            pltpu.SemaphoreType.DMA((NBUF,)),
            pltpu.SemaphoreType.DMA((NBUF,)),
        ],
        compiler_params=pltpu.CompilerParams(use_tc_tiling_on_sc=False),
    )


_agg_w = _make_agg(True)
_agg_u = _make_agg(False)

RB = 512
GRID = ND // RB
RH = 80


def _k1_body(x_ref, w_ref, p0_ref, p1_ref, o_ref):
    dinv = lax.rsqrt(p0_ref[...] + p1_ref[...] + 1.0)
    o_ref[...] = jnp.dot(
        x_ref[...], w_ref[...], preferred_element_type=jnp.float32) * dinv


def _mid_body(z0_ref, z1_ref, y_ref, pp0_ref, pp1_ref, nn0_ref, nn1_ref,
              b_ref, w_ref, o_ref):
    dinvp = lax.rsqrt(pp0_ref[...] + pp1_ref[...] + 1.0)
    h = jnp.maximum(
        (z0_ref[0] + z1_ref[0] + y_ref[...]) * dinvp + b_ref[...], 0.0)
    dinvn = lax.rsqrt(nn0_ref[...] + nn1_ref[...] + 1.0)
    o_ref[...] = jnp.dot(
        h, w_ref[...], preferred_element_type=jnp.float32) * dinvn


def _head_body(z0_ref, z1_ref, y_ref, cp0_ref, cp1_ref, b5_ref,
               m0_ref, m1_ref, wp_ref, bp_ref, wv_ref, bv_ref,
               p_ref, v_ref):
    dinv2 = lax.rsqrt(cp0_ref[...] + cp1_ref[...] + 1.0)
    h = jnp.maximum(
        (z0_ref[0] + z1_ref[0] + y_ref[...]) * dinv2 + b5_ref[...], 0.0)
    msum = m0_ref[...] + m1_ref[...]
    maskf = jnp.where(msum > 0.0, 1.0, 0.0)
    anyrow = jnp.any(msum > 0.0)

    @pl.when(anyrow)
    def _():
        zp = jnp.dot(h, wp_ref[...], preferred_element_type=jnp.float32)
        zp = zp + bp_ref[...]
        zmax = jnp.max(zp, axis=1, keepdims=True)
        ez = jnp.exp(zp - zmax)
        ssum = jnp.sum(ez, axis=1, keepdims=True)
        p_ref[...] = ez * (maskf / ssum)
        zv = jnp.dot(h, wv_ref[...], preferred_element_type=jnp.float32)
        v_ref[...] = (zv + bv_ref[...]) * maskf

    @pl.when(jnp.logical_not(anyrow))
    def _():
        p_ref[...] = jnp.zeros((RH, O), jnp.float32)
        v_ref[...] = jnp.zeros((RH, O), jnp.float32)


def _row_spec(rows, cols):
    return pl.BlockSpec((rows, cols), lambda i: (i, 0))


def _const_spec(rows, cols):
    return pl.BlockSpec((rows, cols), lambda i: (0, 0))


def kernel(x, edge_index, edge_weight, current_node,
           W1, b1, W2, b2, W3, b3, W4, b4, W5, b5, Wp, bp, Wv, bv):
    src = edge_index[0]
    dst = edge_index[1]
    pad = EPAD - E
    srcp = jnp.concatenate(
        [src, jnp.full((pad,), N, jnp.int32)]).reshape(ECH, CHUNK)
    dstp = jnp.concatenate(
        [dst, jnp.full((pad,), N, jnp.int32)]).reshape(ECH, CHUNK)
    ewp = jnp.concatenate(
        [edge_weight, jnp.zeros((pad,), jnp.float32)]).reshape(ECH, CHUNK)
    cur16 = jnp.full((16,), current_node, jnp.int32)
    z1d = jnp.zeros((ND,), jnp.float32)
    z2d = jnp.zeros((ND, H), jnp.float32)
    x_p = jnp.pad(x, ((0, ND - N), (0, 0)))

    degp, cntp, mskp = _prep(srcp, dstp, ewp, cur16, z1d)
    d10 = degp[0].reshape(ND, 1)
    d11 = degp[1].reshape(ND, 1)
    c0 = cntp[0].reshape(ND, 1)
    c1 = cntp[1].reshape(ND, 1)
    m0 = mskp[0].reshape(ND, 1)
    m1 = mskp[1].reshape(ND, 1)

    y1 = pl.pallas_call(
        _k1_body,
        grid=(GRID,),
        in_specs=[
            _row_spec(RB, D_IN),
            _const_spec(D_IN, H),
            _row_spec(RB, 1),
            _row_spec(RB, 1),
        ],
        out_specs=_row_spec(RB, H),
        out_shape=jax.ShapeDtypeStruct((ND, H), jnp.float32),
    )(x_p, W1, d10, d11)

    def mid(zp, y_prev, pp0, pp1, b_prev, w_next):
        return pl.pallas_call(
            _mid_body,
            grid=(GRID,),
            in_specs=[
                pl.BlockSpec((1, RB, H), lambda i: (0, i, 0)),
                pl.BlockSpec((1, RB, H), lambda i: (1, i, 0)),
                _row_spec(RB, H),
                _row_spec(RB, 1),
                _row_spec(RB, 1),
                _row_spec(RB, 1),
                _row_spec(RB, 1),
                _const_spec(1, H),
                _const_spec(H, H),
            ],
            out_specs=_row_spec(RB, H),
            out_shape=jax.ShapeDtypeStruct((ND, H), jnp.float32),
        )(zp, zp, y_prev, pp0, pp1, c0, c1, b_prev.reshape(1, H), w_next)

    zp1 = _agg_w(srcp, dstp, ewp, y1, z2d)
    y2 = mid(zp1, y1, d10, d11, b1, W2)
    zp2 = _agg_u(srcp, dstp, ewp, y2, z2d)
    y3 = mid(zp2, y2, c0, c1, b2, W3)
    zp3 = _agg_u(srcp, dstp, ewp, y3, z2d)
    y4 = mid(zp3, y3, c0, c1, b3, W4)
    zp4 = _agg_u(srcp, dstp, ewp, y4, z2d)
    y5 = mid(zp4, y4, c0, c1, b4, W5)
    zp5 = _agg_u(srcp, dstp, ewp, y5, z2d)

    p, v = pl.pallas_call(
        _head_body,
        grid=(N // RH,),
        in_specs=[
            pl.BlockSpec((1, RH, H), lambda i: (0, i, 0)),
            pl.BlockSpec((1, RH, H), lambda i: (1, i, 0)),
            _row_spec(RH, H),
            _row_spec(RH, 1),
            _row_spec(RH, 1),
            _const_spec(1, H),
            _row_spec(RH, 1),
            _row_spec(RH, 1),
            _const_spec(H, O),
            _const_spec(1, O),
            _const_spec(H, O),
            _const_spec(1, O),
        ],
        out_specs=[_row_spec(RH, O), _row_spec(RH, O)],
        out_shape=[
            jax.ShapeDtypeStruct((N, O), jnp.float32),
            jax.ShapeDtypeStruct((N, O), jnp.float32),
        ],
    )(zp5, zp5, y5, c0, c1, b5.reshape(1, H), m0, m1,
      Wp, bp.reshape(1, O), Wv, bv.reshape(1, O))
    return (p, v)


# speedup vs baseline: 22.4383x; 1.0426x over previous
import functools

import jax
import jax.numpy as jnp
from jax import lax
from jax.experimental import pallas as pl
from jax.experimental.pallas import tpu as pltpu
from jax.experimental.pallas import tpu_sc as plsc

N = 10000
E = 160000
D_IN = 256
H = 32
O = 10000

NC = 2
NS = 16
NW = NC * NS
CHUNK = 128
KCH = 40
EPW = KCH * CHUNK
EPAD = EPW * NW
ECH = EPAD // CHUNK
ND = 10240
RPT = ND // NS
NBUF = 8

_mesh = plsc.VectorSubcoreMesh(
    core_axis_name="c", subcore_axis_name="s", num_cores=NC, num_subcores=NS
)


def _prep_body(src_h, dst_h, ew_h, cur_h, z1_h, degp, cntp, mskp,
               sbuf, dbuf, ebuf, cbuf, obuf, hbuf, accd, accc, accm):
    c = lax.axis_index("c")
    s = lax.axis_index("s")
    wid = c * NS + s
    base = wid * KCH
    pltpu.sync_copy(src_h.at[pl.ds(base, KCH)], sbuf)
    pltpu.sync_copy(dst_h.at[pl.ds(base, KCH)], dbuf)
    pltpu.sync_copy(ew_h.at[pl.ds(base, KCH)], ebuf)
    pltpu.sync_copy(cur_h, cbuf)
    off = s * RPT
    pltpu.sync_copy(z1_h.at[pl.ds(off, RPT)], accd.at[pl.ds(off, RPT)])
    pltpu.sync_copy(z1_h.at[pl.ds(off, RPT)], accc.at[pl.ds(off, RPT)])
    pltpu.sync_copy(z1_h.at[pl.ds(off, RPT)], accm.at[pl.ds(off, RPT)])

    def fill_ones(k, carry):
        obuf[pl.ds(k * 16, 16)] = jnp.full((16,), 1.0, jnp.float32)
        return carry

    lax.fori_loop(0, CHUNK // 16, fill_ones, 0)
    plsc.subcore_barrier()
    cv = cbuf[...]

    def chunk(j, carry):
        def hk(k, inner):
            s16 = sbuf[j, pl.ds(k * 16, 16)]
            hbuf[pl.ds(k * 16, 16)] = jnp.where(
                s16 == cv, jnp.full((16,), 1.0, jnp.float32),
                jnp.full((16,), 0.0, jnp.float32))
            return inner

        lax.fori_loop(0, CHUNK // 16, hk, 0)
        pltpu.sync_copy(ebuf.at[j], accd.at[dbuf.at[j]], add=True)
        pltpu.sync_copy(obuf, accc.at[dbuf.at[j]], add=True)
        pltpu.sync_copy(hbuf, accm.at[dbuf.at[j]], add=True)
        return carry

    lax.fori_loop(0, KCH, chunk, 0)
    plsc.subcore_barrier()
    pltpu.sync_copy(accd.at[pl.ds(off, RPT)], degp.at[c, pl.ds(off, RPT)])
    pltpu.sync_copy(accc.at[pl.ds(off, RPT)], cntp.at[c, pl.ds(off, RPT)])
    pltpu.sync_copy(accm.at[pl.ds(off, RPT)], mskp.at[c, pl.ds(off, RPT)])


_prep = pl.kernel(
    _prep_body,
    out_type=(
        jax.ShapeDtypeStruct((NC, ND), jnp.float32),
        jax.ShapeDtypeStruct((NC, ND), jnp.float32),
        jax.ShapeDtypeStruct((NC, ND), jnp.float32),
    ),
    mesh=_mesh,
    scratch_types=[
        pltpu.VMEM((KCH, CHUNK), jnp.int32),
        pltpu.VMEM((KCH, CHUNK), jnp.int32),
        pltpu.VMEM((KCH, CHUNK), jnp.float32),
        pltpu.VMEM((16,), jnp.int32),
        pltpu.VMEM((CHUNK,), jnp.float32),
        pltpu.VMEM((CHUNK,), jnp.float32),
        pltpu.VMEM_SHARED((ND,), jnp.float32),
        pltpu.VMEM_SHARED((ND,), jnp.float32),
        pltpu.VMEM_SHARED((ND,), jnp.float32),
    ],
)


def _agg_body(weighted, src_h, dst_h, ew_h, y_h, z2_h, outp,
              sbuf, dbuf, ebuf, rows, ystage, acc, gsems, ssems):
    c = lax.axis_index("c")
    s = lax.axis_index("s")
    wid = c * NS + s
    base = wid * KCH
    pltpu.sync_copy(src_h.at[pl.ds(base, KCH)], sbuf)
    pltpu.sync_copy(dst_h.at[pl.ds(base, KCH)], dbuf)
    if weighted:
        pltpu.sync_copy(ew_h.at[pl.ds(base, KCH)], ebuf)
    off = s * RPT
    pltpu.sync_copy(z2_h.at[pl.ds(off, RPT)], acc.at[pl.ds(off, RPT)])
    pltpu.sync_copy(y_h.at[pl.ds(off, RPT)], ystage.at[pl.ds(off, RPT)])
    plsc.subcore_barrier()
    for b in range(NBUF):
        pltpu.async_copy(ystage.at[sbuf.at[b]], rows.at[b], gsems.at[b])

    for i in range(KCH):
        b = i % NBUF
        pltpu.make_async_copy(
            ystage.at[sbuf.at[i]], rows.at[b], gsems.at[b]).wait()
        if weighted:
            def esc(k, inner, i=i, b=b):
                wv = ebuf[i, pl.ds(k * 16, 16)]
                for m in range(16):
                    r = k * 16 + m
                    w = wv[m]
                    rows[b, r, pl.ds(0, 16)] = rows[b, r, pl.ds(0, 16)] * w
                    rows[b, r, pl.ds(16, 16)] = rows[b, r, pl.ds(16, 16)] * w
                return inner

            lax.fori_loop(0, CHUNK // 16, esc, 0)
        pltpu.async_copy(rows.at[b], acc.at[dbuf.at[i]], ssems.at[b], add=True)
        ir = i + NBUF // 2
        if NBUF <= ir < KCH:
            br = ir % NBUF
            pltpu.make_async_copy(
                rows.at[br], acc.at[dbuf.at[ir - NBUF]], ssems.at[br]).wait()
            pltpu.async_copy(ystage.at[sbuf.at[ir]], rows.at[br], gsems.at[br])

    for b in range(NBUF):
        i_last = KCH - NBUF + b
        pltpu.make_async_copy(
            rows.at[b], acc.at[dbuf.at[i_last]], ssems.at[b]).wait()
    plsc.subcore_barrier()
    pltpu.sync_copy(acc.at[pl.ds(off, RPT)], outp.at[c, pl.ds(off, RPT)])


def _make_agg(weighted):
    return pl.kernel(
        functools.partial(_agg_body, weighted),
        out_type=jax.ShapeDtypeStruct((NC, ND, H), jnp.float32),
        mesh=_mesh,
        scratch_types=[
            pltpu.VMEM((KCH, CHUNK), jnp.int32),
            pltpu.VMEM((KCH, CHUNK), jnp.int32),
            pltpu.VMEM((KCH, CHUNK), jnp.float32),
            pltpu.VMEM((NBUF, CHUNK, H), jnp.float32),
            pltpu.VMEM_SHARED((ND, H), jnp.float32),
            pltpu.VMEM_SHARED((ND, H), jnp.float32),
            pltpu.SemaphoreType.DMA((NBUF,)),
            pltpu.SemaphoreType.DMA((NBUF,)),
        ],
        compiler_params=pltpu.CompilerParams(use_tc_tiling_on_sc=False),
    )


_agg_w = _make_agg(True)
_agg_u = _make_agg(False)

RB = 512
GRID = ND // RB
RH = 80


def _mm1_body(x_ref, w_ref, o_ref):
    o_ref[...] = jnp.dot(
        x_ref[...], w_ref[...], preferred_element_type=jnp.float32)


def _nodeprep_body(xw_ref, dg_ref, cn_ref, mk_ref,
                   y1_ref, dm1_ref, dm2_ref, mm_ref):
    dg = dg_ref[...]
    cn = cn_ref[...]
    mk = mk_ref[...]
    d1 = lax.rsqrt(dg[0] + dg[1] + 1.0)
    d2 = lax.rsqrt(cn[0] + cn[1] + 1.0)
    mf = jnp.where(mk[0] + mk[1] > 0.0, 1.0, 0.0)
    dm1 = jnp.broadcast_to(d1[:, None], (RB, H))
    dm2 = jnp.broadcast_to(d2[:, None], (RB, H))
    dm1_ref[...] = dm1
    dm2_ref[...] = dm2
    mm_ref[...] = jnp.broadcast_to(mf[:, None], (RB, H))
    y1_ref[...] = xw_ref[...] * dm1


def _mid_body(z0_ref, z1_ref, y_ref, dp_ref, dn_ref, b_ref, w_ref, o_ref):
    h = jnp.maximum(
        (z0_ref[0] + z1_ref[0] + y_ref[...]) * dp_ref[...] + b_ref[...], 0.0)
    o_ref[...] = jnp.dot(
        h, w_ref[...], preferred_element_type=jnp.float32) * dn_ref[...]


def _head_body(z0_ref, z1_ref, y_ref, dn_ref, b5_ref, mm_ref,
               wp_ref, bp_ref, wv_ref, bv_ref, p_ref, v_ref):
    h = jnp.maximum(
        (z0_ref[0] + z1_ref[0] + y_ref[...]) * dn_ref[...] + b5_ref[...], 0.0)
    maskf = mm_ref[:, 0:1]
    anyrow = jnp.any(maskf > 0.0)

    @pl.when(anyrow)
    def _():
        zp = jnp.dot(h, wp_ref[...], preferred_element_type=jnp.float32)
        zp = zp + bp_ref[...]
        zmax = jnp.max(zp, axis=1, keepdims=True)
        ez = jnp.exp(zp - zmax)
        ssum = jnp.sum(ez, axis=1, keepdims=True)
        p_ref[...] = ez * (maskf / ssum)
        zv = jnp.dot(h, wv_ref[...], preferred_element_type=jnp.float32)
        v_ref[...] = (zv + bv_ref[...]) * maskf

    @pl.when(jnp.logical_not(anyrow))
    def _():
        p_ref[...] = jnp.zeros((RH, O), jnp.float32)
        v_ref[...] = jnp.zeros((RH, O), jnp.float32)


def _row_spec(rows, cols):
    return pl.BlockSpec((rows, cols), lambda i: (i, 0))


def _const_spec(rows, cols):
    return pl.BlockSpec((rows, cols), lambda i: (0, 0))


def kernel(x, edge_index, edge_weight, current_node,
           W1, b1, W2, b2, W3, b3, W4, b4, W5, b5, Wp, bp, Wv, bv):
    src = edge_index[0]
    dst = edge_index[1]
    pad = EPAD - E
    srcp = jnp.concatenate(
        [src, jnp.full((pad,), N, jnp.int32)]).reshape(ECH, CHUNK)
    dstp = jnp.concatenate(
        [dst, jnp.full((pad,), N, jnp.int32)]).reshape(ECH, CHUNK)
    ewp = jnp.concatenate(
        [edge_weight, jnp.zeros((pad,), jnp.float32)]).reshape(ECH, CHUNK)
    cur16 = jnp.full((16,), current_node, jnp.int32)
    z1d = jnp.zeros((ND,), jnp.float32)
    z2d = jnp.zeros((ND, H), jnp.float32)
    x_p = jnp.pad(x, ((0, ND - N), (0, 0)))

    xw1 = pl.pallas_call(
        _mm1_body,
        grid=(GRID,),
        in_specs=[_row_spec(RB, D_IN), _const_spec(D_IN, H)],
        out_specs=_row_spec(RB, H),
        out_shape=jax.ShapeDtypeStruct((ND, H), jnp.float32),
    )(x_p, W1)

    degp, cntp, mskp = _prep(srcp, dstp, ewp, cur16, z1d)

    nc_spec = pl.BlockSpec((NC, RB), lambda i: (0, i))
    y1, dm1, dm2, maskm = pl.pallas_call(
        _nodeprep_body,
        grid=(GRID,),
        in_specs=[_row_spec(RB, H), nc_spec, nc_spec, nc_spec],
        out_specs=[_row_spec(RB, H)] * 4,
        out_shape=[jax.ShapeDtypeStruct((ND, H), jnp.float32)] * 4,
    )(xw1, degp, cntp, mskp)

    def mid(zp, y_prev, dmp, b_prev, w_next):
        return pl.pallas_call(
            _mid_body,
            grid=(GRID,),
            in_specs=[
                pl.BlockSpec((1, RB, H), lambda i: (0, i, 0)),
                pl.BlockSpec((1, RB, H), lambda i: (1, i, 0)),
                _row_spec(RB, H),
                _row_spec(RB, H),
                _row_spec(RB, H),
                _const_spec(1, H),
                _const_spec(H, H),
            ],
            out_specs=_row_spec(RB, H),
            out_shape=jax.ShapeDtypeStruct((ND, H), jnp.float32),
        )(zp, zp, y_prev, dmp, dm2, b_prev.reshape(1, H), w_next)

    zp1 = _agg_w(srcp, dstp, ewp, y1, z2d)
    y2 = mid(zp1, y1, dm1, b1, W2)
    zp2 = _agg_u(srcp, dstp, ewp, y2, z2d)
    y3 = mid(zp2, y2, dm2, b2, W3)
    zp3 = _agg_u(srcp, dstp, ewp, y3, z2d)
    y4 = mid(zp3, y3, dm2, b3, W4)
    zp4 = _agg_u(srcp, dstp, ewp, y4, z2d)
    y5 = mid(zp4, y4, dm2, b4, W5)
    zp5 = _agg_u(srcp, dstp, ewp, y5, z2d)

    p, v = pl.pallas_call(
        _head_body,
        grid=(N // RH,),
        in_specs=[
            pl.BlockSpec((1, RH, H), lambda i: (0, i, 0)),
            pl.BlockSpec((1, RH, H), lambda i: (1, i, 0)),
            _row_spec(RH, H),
            _row_spec(RH, H),
            _const_spec(1, H),
            _row_spec(RH, H),
            _const_spec(H, O),
            _const_spec(1, O),
            _const_spec(H, O),
            _const_spec(1, O),
        ],
        out_specs=[_row_spec(RH, O), _row_spec(RH, O)],
        out_shape=[
            jax.ShapeDtypeStruct((N, O), jnp.float32),
            jax.ShapeDtypeStruct((N, O), jnp.float32),
        ],
    )(zp5, zp5, y5, dm2, b5.reshape(1, H), maskm,
      Wp, bp.reshape(1, O), Wv, bv.reshape(1, O))
    return (p, v)


# speedup vs baseline: 23.5529x; 1.0497x over previous
import functools

import jax
import jax.numpy as jnp
from jax import lax
from jax.experimental import pallas as pl
from jax.experimental.pallas import tpu as pltpu
from jax.experimental.pallas import tpu_sc as plsc

N = 10000
E = 160000
D_IN = 256
H = 32
O = 10000

NC = 2
NS = 16
NW = NC * NS
CHUNK = 128
KCH = 40
EPW = KCH * CHUNK
EPAD = EPW * NW
ECH = EPAD // CHUNK
ND = 10240
RPT = ND // NS
NBUF = 8

_mesh = plsc.VectorSubcoreMesh(
    core_axis_name="c", subcore_axis_name="s", num_cores=NC, num_subcores=NS
)


def _prep_body(src_h, dst_h, ew_h, cur_h, z1_h, degp, cntp, mskp,
               sbuf, dbuf, ebuf, cbuf, obuf, hbuf, accd, accc, accm):
    c = lax.axis_index("c")
    s = lax.axis_index("s")
    wid = c * NS + s
    base = wid * KCH
    pltpu.sync_copy(src_h.at[pl.ds(base, KCH)], sbuf)
    pltpu.sync_copy(dst_h.at[pl.ds(base, KCH)], dbuf)
    pltpu.sync_copy(ew_h.at[pl.ds(base, KCH)], ebuf)
    pltpu.sync_copy(cur_h, cbuf)
    off = s * RPT
    pltpu.sync_copy(z1_h.at[pl.ds(off, RPT)], accd.at[pl.ds(off, RPT)])
    pltpu.sync_copy(z1_h.at[pl.ds(off, RPT)], accc.at[pl.ds(off, RPT)])
    pltpu.sync_copy(z1_h.at[pl.ds(off, RPT)], accm.at[pl.ds(off, RPT)])

    def fill_ones(k, carry):
        obuf[pl.ds(k * 16, 16)] = jnp.full((16,), 1.0, jnp.float32)
        return carry

    lax.fori_loop(0, CHUNK // 16, fill_ones, 0)
    plsc.subcore_barrier()
    cv = cbuf[...]

    def chunk(j, carry):
        def hk(k, inner):
            s16 = sbuf[j, pl.ds(k * 16, 16)]
            hbuf[pl.ds(k * 16, 16)] = jnp.where(
                s16 == cv, jnp.full((16,), 1.0, jnp.float32),
                jnp.full((16,), 0.0, jnp.float32))
            return inner

        lax.fori_loop(0, CHUNK // 16, hk, 0)
        pltpu.sync_copy(ebuf.at[j], accd.at[dbuf.at[j]], add=True)
        pltpu.sync_copy(obuf, accc.at[dbuf.at[j]], add=True)
        pltpu.sync_copy(hbuf, accm.at[dbuf.at[j]], add=True)
        return carry

    lax.fori_loop(0, KCH, chunk, 0)
    plsc.subcore_barrier()
    pltpu.sync_copy(accd.at[pl.ds(off, RPT)], degp.at[c, pl.ds(off, RPT)])
    pltpu.sync_copy(accc.at[pl.ds(off, RPT)], cntp.at[c, pl.ds(off, RPT)])
    pltpu.sync_copy(accm.at[pl.ds(off, RPT)], mskp.at[c, pl.ds(off, RPT)])


_prep = pl.kernel(
    _prep_body,
    out_type=(
        jax.ShapeDtypeStruct((NC, ND), jnp.float32),
        jax.ShapeDtypeStruct((NC, ND), jnp.float32),
        jax.ShapeDtypeStruct((NC, ND), jnp.float32),
    ),
    mesh=_mesh,
    scratch_types=[
        pltpu.VMEM((KCH, CHUNK), jnp.int32),
        pltpu.VMEM((KCH, CHUNK), jnp.int32),
        pltpu.VMEM((KCH, CHUNK), jnp.float32),
        pltpu.VMEM((16,), jnp.int32),
        pltpu.VMEM((CHUNK,), jnp.float32),
        pltpu.VMEM((CHUNK,), jnp.float32),
        pltpu.VMEM_SHARED((ND,), jnp.float32),
        pltpu.VMEM_SHARED((ND,), jnp.float32),
        pltpu.VMEM_SHARED((ND,), jnp.float32),
    ],
)


def _agg_body(weighted, src_h, dst_h, ew_h, y_h, z2_h, outp,
              sbuf, dbuf, ebuf, rows, ystage, acc, gsems, ssems):
    c = lax.axis_index("c")
    s = lax.axis_index("s")
    wid = c * NS + s
    base = wid * KCH
    pltpu.sync_copy(src_h.at[pl.ds(base, KCH)], sbuf)
    pltpu.sync_copy(dst_h.at[pl.ds(base, KCH)], dbuf)
    if weighted:
        pltpu.sync_copy(ew_h.at[pl.ds(base, KCH)], ebuf)
    off = s * RPT
    pltpu.sync_copy(z2_h.at[pl.ds(off, RPT)], acc.at[pl.ds(off, RPT)])
    pltpu.sync_copy(y_h.at[pl.ds(off, RPT)], ystage.at[pl.ds(off, RPT)])
    plsc.subcore_barrier()
    for b in range(NBUF):
        pltpu.async_copy(ystage.at[sbuf.at[b]], rows.at[b], gsems.at[b])

    for i in range(KCH):
        b = i % NBUF
        pltpu.make_async_copy(
            ystage.at[sbuf.at[i]], rows.at[b], gsems.at[b]).wait()
        if weighted:
            def esc(k, inner, i=i, b=b):
                wv = ebuf[i, pl.ds(k * 16, 16)]
                for m in range(16):
                    r = k * 16 + m
                    w = wv[m]
                    rows[b, r, pl.ds(0, 16)] = rows[b, r, pl.ds(0, 16)] * w
                    rows[b, r, pl.ds(16, 16)] = rows[b, r, pl.ds(16, 16)] * w
                return inner

            lax.fori_loop(0, CHUNK // 16, esc, 0)
        pltpu.async_copy(rows.at[b], acc.at[dbuf.at[i]], ssems.at[b], add=True)
        ir = i + NBUF // 2
        if NBUF <= ir < KCH:
            br = ir % NBUF
            pltpu.make_async_copy(
                rows.at[br], acc.at[dbuf.at[ir - NBUF]], ssems.at[br]).wait()
            pltpu.async_copy(ystage.at[sbuf.at[ir]], rows.at[br], gsems.at[br])

    for b in range(NBUF):
        i_last = KCH - NBUF + b
        pltpu.make_async_copy(
            rows.at[b], acc.at[dbuf.at[i_last]], ssems.at[b]).wait()
    plsc.subcore_barrier()
    pltpu.sync_copy(acc.at[pl.ds(off, RPT)], outp.at[c, pl.ds(off, RPT)])


def _make_agg(weighted):
    return pl.kernel(
        functools.partial(_agg_body, weighted),
        out_type=jax.ShapeDtypeStruct((NC, ND, H), jnp.float32),
        mesh=_mesh,
        scratch_types=[
            pltpu.VMEM((KCH, CHUNK), jnp.int32),
            pltpu.VMEM((KCH, CHUNK), jnp.int32),
            pltpu.VMEM((KCH, CHUNK), jnp.float32),
            pltpu.VMEM((NBUF, CHUNK, H), jnp.float32),
            pltpu.VMEM_SHARED((ND, H), jnp.float32),
            pltpu.VMEM_SHARED((ND, H), jnp.float32),
            pltpu.SemaphoreType.DMA((NBUF,)),
            pltpu.SemaphoreType.DMA((NBUF,)),
        ],
        compiler_params=pltpu.CompilerParams(use_tc_tiling_on_sc=False),
    )


_agg_w = _make_agg(True)
_agg_u = _make_agg(False)

RB = 1024
GRID = ND // RB
RH = 80


def _mm1_body(x_ref, w_ref, o_ref):
    o_ref[...] = jnp.dot(
        x_ref[...], w_ref[...], preferred_element_type=jnp.float32)


def _nodeprep_body(xw_ref, dg_ref, cn_ref, mk_ref,
                   y1_ref, dm1_ref, dm2_ref, mm_ref):
    dg = dg_ref[...]
    cn = cn_ref[...]
    mk = mk_ref[...]
    d1 = lax.rsqrt(dg[0] + dg[1] + 1.0)
    d2 = lax.rsqrt(cn[0] + cn[1] + 1.0)
    mf = jnp.where(mk[0] + mk[1] > 0.0, 1.0, 0.0)
    dm1 = jnp.broadcast_to(d1[:, None], (RB, H))
    dm2 = jnp.broadcast_to(d2[:, None], (RB, H))
    dm1_ref[...] = dm1
    dm2_ref[...] = dm2
    mm_ref[...] = jnp.broadcast_to(mf[:, None], (RB, H))
    y1_ref[...] = xw_ref[...] * dm1


def _mid_body(z0_ref, z1_ref, y_ref, dp_ref, dn_ref, b_ref, w_ref, o_ref):
    h = jnp.maximum(
        (z0_ref[0] + z1_ref[0] + y_ref[...]) * dp_ref[...] + b_ref[...], 0.0)
    o_ref[...] = jnp.dot(
        h, w_ref[...], preferred_element_type=jnp.float32) * dn_ref[...]


def _head_body(z0_ref, z1_ref, y_ref, dn_ref, b5_ref, mm_ref,
               wp_ref, bp_ref, wv_ref, bv_ref, p_ref, v_ref):
    h = jnp.maximum(
        (z0_ref[0] + z1_ref[0] + y_ref[...]) * dn_ref[...] + b5_ref[...], 0.0)
    maskf = mm_ref[:, 0:1]
    anyrow = jnp.any(maskf > 0.0)

    @pl.when(anyrow)
    def _():
        zp = jnp.dot(h, wp_ref[...], preferred_element_type=jnp.float32)
        zp = zp + bp_ref[...]
        zmax = jnp.max(zp, axis=1, keepdims=True)
        ez = jnp.exp(zp - zmax)
        ssum = jnp.sum(ez, axis=1, keepdims=True)
        p_ref[...] = ez * (maskf / ssum)
        zv = jnp.dot(h, wv_ref[...], preferred_element_type=jnp.float32)
        v_ref[...] = (zv + bv_ref[...]) * maskf

    @pl.when(jnp.logical_not(anyrow))
    def _():
        p_ref[...] = jnp.zeros((RH, O), jnp.float32)
        v_ref[...] = jnp.zeros((RH, O), jnp.float32)


def _row_spec(rows, cols):
    return pl.BlockSpec((rows, cols), lambda i: (i, 0))


def _const_spec(rows, cols):
    return pl.BlockSpec((rows, cols), lambda i: (0, 0))


def kernel(x, edge_index, edge_weight, current_node,
           W1, b1, W2, b2, W3, b3, W4, b4, W5, b5, Wp, bp, Wv, bv):
    src = edge_index[0]
    dst = edge_index[1]
    pad = EPAD - E
    srcp = jnp.concatenate(
        [src, jnp.full((pad,), N, jnp.int32)]).reshape(ECH, CHUNK)
    dstp = jnp.concatenate(
        [dst, jnp.full((pad,), N, jnp.int32)]).reshape(ECH, CHUNK)
    ewp = jnp.concatenate(
        [edge_weight, jnp.zeros((pad,), jnp.float32)]).reshape(ECH, CHUNK)
    cur16 = jnp.full((16,), current_node, jnp.int32)
    z1d = jnp.zeros((ND,), jnp.float32)
    z2d = jnp.zeros((ND, H), jnp.float32)
    x_p = jnp.pad(x, ((0, ND - N), (0, 0)))

    xw1 = pl.pallas_call(
        _mm1_body,
        grid=(GRID,),
        in_specs=[_row_spec(RB, D_IN), _const_spec(D_IN, H)],
        out_specs=_row_spec(RB, H),
        out_shape=jax.ShapeDtypeStruct((ND, H), jnp.float32),
    )(x_p, W1)

    degp, cntp, mskp = _prep(srcp, dstp, ewp, cur16, z1d)

    nc_spec = pl.BlockSpec((NC, RB), lambda i: (0, i))
    y1, dm1, dm2, maskm = pl.pallas_call(
        _nodeprep_body,
        grid=(GRID,),
        in_specs=[_row_spec(RB, H), nc_spec, nc_spec, nc_spec],
        out_specs=[_row_spec(RB, H)] * 4,
        out_shape=[jax.ShapeDtypeStruct((ND, H), jnp.float32)] * 4,
    )(xw1, degp, cntp, mskp)

    def mid(zp, y_prev, dmp, b_prev, w_next):
        return pl.pallas_call(
            _mid_body,
            grid=(GRID,),
            in_specs=[
                pl.BlockSpec((1, RB, H), lambda i: (0, i, 0)),
                pl.BlockSpec((1, RB, H), lambda i: (1, i, 0)),
                _row_spec(RB, H),
                _row_spec(RB, H),
                _row_spec(RB, H),
                _const_spec(1, H),
                _const_spec(H, H),
            ],
            out_specs=_row_spec(RB, H),
            out_shape=jax.ShapeDtypeStruct((ND, H), jnp.float32),
        )(zp, zp, y_prev, dmp, dm2, b_prev.reshape(1, H), w_next)

    zp1 = _agg_w(srcp, dstp, ewp, y1, z2d)
    y2 = mid(zp1, y1, dm1, b1, W2)
    zp2 = _agg_u(srcp, dstp, ewp, y2, z2d)
    y3 = mid(zp2, y2, dm2, b2, W3)
    zp3 = _agg_u(srcp, dstp, ewp, y3, z2d)
    y4 = mid(zp3, y3, dm2, b3, W4)
    zp4 = _agg_u(srcp, dstp, ewp, y4, z2d)
    y5 = mid(zp4, y4, dm2, b4, W5)
    zp5 = _agg_u(srcp, dstp, ewp, y5, z2d)

    p, v = pl.pallas_call(
        _head_body,
        grid=(N // RH,),
        in_specs=[
            pl.BlockSpec((1, RH, H), lambda i: (0, i, 0)),
            pl.BlockSpec((1, RH, H), lambda i: (1, i, 0)),
            _row_spec(RH, H),
            _row_spec(RH, H),
            _const_spec(1, H),
            _row_spec(RH, H),
            _const_spec(H, O),
            _const_spec(1, O),
            _const_spec(H, O),
            _const_spec(1, O),
        ],
        out_specs=[_row_spec(RH, O), _row_spec(RH, O)],
        out_shape=[
            jax.ShapeDtypeStruct((N, O), jnp.float32),
            jax.ShapeDtypeStruct((N, O), jnp.float32),
        ],
    )(zp5, zp5, y5, dm2, b5.reshape(1, H), maskm,
      Wp, bp.reshape(1, O), Wv, bv.reshape(1, O))
    return (p, v)
